# baseline (speedup 1.0000x reference)
import functools
import math

import jax
import jax.numpy as jnp
from jax import lax
from jax.experimental import pallas as pl
from jax.experimental.pallas import tpu as pltpu
from jax.experimental.pallas import tpu_sc as plsc

B = 16384
L = 50
D = 32
NCAT = 128
VOCAB = 1000000
NC, NS = 2, 16
NW = NC * NS
RPW = B // NW
C = 8
NCH = RPW // C
GI = C * L
IPW = RPW * L

S = 1 << 18
SMASK = S - 1
SSHIFT = 18


def _tr4_body(t0, t1, t2, t3, o_ref):
    s = lax.broadcasted_iota(jnp.int32, (D, D), 0)
    dcol = lax.broadcasted_iota(jnp.int32, (D, D), 1)
    perm = jnp.where(s < D // 2, 2 * s, 2 * (s - D // 2) + 1)
    pmat = (dcol == perm).astype(jnp.float32)

    def tr(t_ref):
        return lax.dot_general(
            t_ref[...], pmat,
            dimension_numbers=(((0,), (0,)), ((), ())),
            preferred_element_type=jnp.float32,
            precision=lax.Precision.DEFAULT,
        ).astype(jnp.bfloat16)

    o_ref[:, 0:D] = tr(t0)
    o_ref[:, D:2 * D] = tr(t1)
    o_ref[:, 2 * D:3 * D] = tr(t2)
    o_ref[:, 3 * D:4 * D] = tr(t3)


def _tc_pack(table_t, blk=2048):
    nblocks = math.ceil(VOCAB / blk)
    off = S // blk

    def imap(j):
        return lambda i: (0, jnp.minimum(i + j * off, nblocks - 1))

    return pl.pallas_call(
        _tr4_body,
        grid=(S // blk,),
        in_specs=[pl.BlockSpec((D, blk), imap(j)) for j in range(4)],
        out_specs=pl.BlockSpec((blk, 4 * D), lambda i: (i, 0)),
        out_shape=jax.ShapeDtypeStruct((S, 4 * D), jnp.bfloat16),
    )(table_t, table_t, table_t, table_t)


def _sc_pool(desc_flat, table_c):
    mesh = plsc.VectorSubcoreMesh(core_axis_name="c", subcore_axis_name="s")

    @functools.partial(
        pl.kernel,
        out_type=jax.ShapeDtypeStruct((B, D), jnp.float32),
        mesh=mesh,
        compiler_params=pltpu.CompilerParams(
            use_tc_tiling_on_sc=False, needs_layout_passes=False
        ),
        scratch_types=[
            pltpu.VMEM((GI,), jnp.int32),
            pltpu.VMEM((GI,), jnp.int32),
            pltpu.VMEM((GI,), jnp.int32),
            pltpu.VMEM((GI,), jnp.int32),
            pltpu.VMEM((GI, 4 * D), jnp.bfloat16),
            pltpu.VMEM((GI, 4 * D), jnp.bfloat16),
            pltpu.VMEM((RPW, D), jnp.float32),
            pltpu.SemaphoreType.DMA,
            pltpu.SemaphoreType.DMA,
            pltpu.SemaphoreType.DMA,
            pltpu.SemaphoreType.DMA,
        ],
    )
    def k(desc_hbm, table_hbm, out_hbm,
          raw0, raw1, g0, g1, rows0, rows1, sums_v,
          siv0, siv1, sg0, sg1):
        cid = lax.axis_index("c")
        sid = lax.axis_index("s")
        wid = sid * NC + cid
        ibase = wid * IPW

        raws = (raw0, raw1)
        gs = (g0, g1)
        rows = (rows0, rows1)
        sivs = (siv0, siv1)
        sgs = (sg0, sg1)

        def fire_idx(c, sub):
            src = desc_hbm.at[pl.ds(ibase + c * GI, GI)]
            pltpu.async_copy(src, raws[sub], sivs[sub])

        def wait_idx(c, sub):
            src = desc_hbm.at[pl.ds(ibase + c * GI, GI)]
            pltpu.make_async_copy(src, raws[sub], sivs[sub]).wait()

        def transform(sub):
            raw, g = raws[sub], gs[sub]

            def body(i, carry):
                v = raw[pl.ds(i * 16, 16)]
                g[pl.ds(i * 16, 16)] = v & SMASK
                return carry

            lax.fori_loop(0, GI // 16, body, 0)

        def fire_gather(c, sub):
            pltpu.async_copy(table_hbm.at[gs[sub]], rows[sub], sgs[sub])

        def wait_gather(c, sub):
            pltpu.make_async_copy(
                table_hbm.at[gs[sub]], rows[sub], sgs[sub]
            ).wait()

        def reduce_chunk(c, sub):
            rbuf, raw = rows[sub], raws[sub]
            offv = []
            for gi in range(GI // 16):
                rv = raw[pl.ds(gi * 16, 16)]
                offv.append((rv >> SSHIFT) * D)

            def off_of(k):
                return offv[k // 16][k % 16]

            def load2(k):
                v = rbuf[k, pl.ds(off_of(k), D)]
                return plsc.unpack(
                    v, format=plsc.PackFormat.INTERLEAVED,
                    preferred_element_type=jnp.float32,
                )

            for b in range(C):
                k0 = b * L
                lo = []
                hi = []
                for gslot in range(5):
                    sl, sh = load2(k0 + gslot)
                    for l in range(gslot + 5, L, 5):
                        a, bb = load2(k0 + l)
                        sl = sl + a
                        sh = sh + bb
                    lo.append(sl)
                    hi.append(sh)
                out_r = c * C + b
                sums_v[out_r, pl.ds(0, 16)] = (lo[0] + lo[1]) + (lo[2] + lo[3]) + lo[4]
                sums_v[out_r, pl.ds(16, 16)] = (hi[0] + hi[1]) + (hi[2] + hi[3]) + hi[4]

        fire_idx(0, 0)
        fire_idx(1, 1)
        wait_idx(0, 0)
        transform(0)
        fire_gather(0, 0)

        def pbody(p, carry):
            for sub in range(2):
                c = p * 2 + sub
                nsub = 1 - sub
                nxt = c + 1

                @pl.when(nxt < NCH)
                def _():
                    wait_idx(nxt, nsub)
                    transform(nsub)
                    fire_gather(nxt, nsub)

                wait_gather(c, sub)
                reduce_chunk(c, sub)
                nxt2 = c + 2

                @pl.when(nxt2 < NCH)
                def _():
                    fire_idx(nxt2, sub)

            return carry

        lax.fori_loop(0, NCH // 2, pbody, 0)
        pltpu.sync_copy(sums_v, out_hbm.at[pl.ds(wid * RPW, RPW)])

    return k(desc_flat, table_c)


def _lin_body(s_ref, a_ref, wm_ref, wa_ref, b_ref, o_ref):
    o_ref[...] = (
        jnp.dot(s_ref[...], wm_ref[...], preferred_element_type=jnp.float32)
        + a_ref[...] * wa_ref[...]
        + b_ref[...]
    )


def _tc_linear(sums, amounts, wm, wa, b2):
    blk = 1024
    return pl.pallas_call(
        _lin_body,
        grid=(B // blk,),
        in_specs=[
            pl.BlockSpec((blk, D), lambda i: (i, 0)),
            pl.BlockSpec((blk, 1), lambda i: (i, 0)),
            pl.BlockSpec((D, NCAT), lambda i: (0, 0)),
            pl.BlockSpec((1, NCAT), lambda i: (0, 0)),
            pl.BlockSpec((1, NCAT), lambda i: (0, 0)),
        ],
        out_specs=pl.BlockSpec((blk, NCAT), lambda i: (i, 0)),
        out_shape=jax.ShapeDtypeStruct((B, NCAT), jnp.float32),
    )(sums, amounts, wm, wa, b2)


def kernel(descriptions, amounts, table, W, b):
    desc_flat = descriptions.reshape(-1)
    table_c = _tc_pack(table.T)
    sums = _sc_pool(desc_flat, table_c)
    wm = W[:D] * (1.0 / L)
    wa = W[D : D + 1]
    b2 = b.reshape(1, NCAT)
    return _tc_linear(sums, amounts, wm, wa, b2)


# speedup vs baseline: 2.1120x; 2.1120x over previous
import functools
import math

import jax
import jax.numpy as jnp
from jax import lax
from jax.experimental import pallas as pl
from jax.experimental.pallas import tpu as pltpu
from jax.experimental.pallas import tpu_sc as plsc

B = 16384
L = 50
D = 32
NCAT = 128
VOCAB = 1000000
NC, NS = 2, 16
NW = NC * NS
RPW = B // NW
C = 8
NCH = RPW // C
GI = C * L
IPW = RPW * L

S = 1 << 18
SMASK = S - 1
SSHIFT = 18


def _tr4_body(t0, t1, t2, t3, o_ref):
    eye = (
        lax.broadcasted_iota(jnp.int32, (4 * D, 4 * D), 0)
        == lax.broadcasted_iota(jnp.int32, (4 * D, 4 * D), 1)
    ).astype(jnp.float32)
    xcat = jnp.concatenate([t0[...], t1[...], t2[...], t3[...]], axis=0)
    o_ref[...] = lax.dot_general(
        xcat, eye,
        dimension_numbers=(((0,), (0,)), ((), ())),
        preferred_element_type=jnp.float32,
        precision=lax.Precision.DEFAULT,
    )


def _tc_pack(table_t, blk=16384):
    nblocks = math.ceil(VOCAB / blk)
    off = S // blk

    def imap(j):
        return lambda i: (0, jnp.minimum(i + j * off, nblocks - 1))

    return pl.pallas_call(
        _tr4_body,
        grid=(S // blk,),
        in_specs=[pl.BlockSpec((D, blk), imap(j)) for j in range(4)],
        out_specs=pl.BlockSpec((blk, 4 * D), lambda i: (i, 0)),
        out_shape=jax.ShapeDtypeStruct((S, 4 * D), jnp.float32),
    )(table_t, table_t, table_t, table_t)


def _sc_pool(desc_flat, table_c):
    mesh = plsc.VectorSubcoreMesh(core_axis_name="c", subcore_axis_name="s")

    @functools.partial(
        pl.kernel,
        out_type=jax.ShapeDtypeStruct((B, D), jnp.float32),
        mesh=mesh,
        compiler_params=pltpu.CompilerParams(use_tc_tiling_on_sc=False),
        scratch_types=[
            pltpu.VMEM((GI,), jnp.int32),
            pltpu.VMEM((GI,), jnp.int32),
            pltpu.VMEM((GI,), jnp.int32),
            pltpu.VMEM((GI,), jnp.int32),
            pltpu.VMEM((GI, 4 * D), jnp.float32),
            pltpu.VMEM((GI, 4 * D), jnp.float32),
            pltpu.VMEM((RPW, D), jnp.float32),
            pltpu.SemaphoreType.DMA,
            pltpu.SemaphoreType.DMA,
            pltpu.SemaphoreType.DMA,
            pltpu.SemaphoreType.DMA,
        ],
    )
    def k(desc_hbm, table_hbm, out_hbm,
          raw0, raw1, g0, g1, rows0, rows1, sums_v,
          siv0, siv1, sg0, sg1):
        cid = lax.axis_index("c")
        sid = lax.axis_index("s")
        wid = sid * NC + cid
        ibase = wid * IPW

        raws = (raw0, raw1)
        gs = (g0, g1)
        rows = (rows0, rows1)
        sivs = (siv0, siv1)
        sgs = (sg0, sg1)

        def fire_idx(c, sub):
            src = desc_hbm.at[pl.ds(ibase + c * GI, GI)]
            pltpu.async_copy(src, raws[sub], sivs[sub])

        def wait_idx(c, sub):
            src = desc_hbm.at[pl.ds(ibase + c * GI, GI)]
            pltpu.make_async_copy(src, raws[sub], sivs[sub]).wait()

        def transform(sub):
            raw, g = raws[sub], gs[sub]

            def body(i, carry):
                v = raw[pl.ds(i * 16, 16)]
                g[pl.ds(i * 16, 16)] = v & SMASK
                return carry

            lax.fori_loop(0, GI // 16, body, 0)

        def fire_gather(c, sub):
            pltpu.async_copy(table_hbm.at[gs[sub]], rows[sub], sgs[sub])

        def wait_gather(c, sub):
            pltpu.make_async_copy(
                table_hbm.at[gs[sub]], rows[sub], sgs[sub]
            ).wait()

        def reduce_chunk(c, sub):
            rbuf, raw = rows[sub], raws[sub]
            offv = []
            for gi in range(GI // 16):
                rv = raw[pl.ds(gi * 16, 16)]
                offv.append((rv >> SSHIFT) * D)

            def off_of(k):
                return offv[k // 16][k % 16]

            for b in range(C):
                k0 = b * L
                lo = []
                hi = []
                for gslot in range(5):
                    k = k0 + gslot
                    off = off_of(k)
                    sl = rbuf[k, pl.ds(off, 16)]
                    sh = rbuf[k, pl.ds(off + 16, 16)]
                    for l in range(gslot + 5, L, 5):
                        k = k0 + l
                        off = off_of(k)
                        sl = sl + rbuf[k, pl.ds(off, 16)]
                        sh = sh + rbuf[k, pl.ds(off + 16, 16)]
                    lo.append(sl)
                    hi.append(sh)
                out_r = c * C + b
                sums_v[out_r, pl.ds(0, 16)] = (lo[0] + lo[1]) + (lo[2] + lo[3]) + lo[4]
                sums_v[out_r, pl.ds(16, 16)] = (hi[0] + hi[1]) + (hi[2] + hi[3]) + hi[4]

        fire_idx(0, 0)
        fire_idx(1, 1)
        wait_idx(0, 0)
        transform(0)
        fire_gather(0, 0)

        def pbody(p, carry):
            for sub in range(2):
                c = p * 2 + sub
                nsub = 1 - sub
                nxt = c + 1

                @pl.when(nxt < NCH)
                def _():
                    wait_idx(nxt, nsub)
                    transform(nsub)
                    fire_gather(nxt, nsub)

                wait_gather(c, sub)
                reduce_chunk(c, sub)
                nxt2 = c + 2

                @pl.when(nxt2 < NCH)
                def _():
                    fire_idx(nxt2, sub)

            return carry

        lax.fori_loop(0, NCH // 2, pbody, 0)
        pltpu.sync_copy(sums_v, out_hbm.at[pl.ds(wid * RPW, RPW)])

    return k(desc_flat, table_c)


def _lin_body(s_ref, a_ref, wm_ref, wa_ref, b_ref, o_ref):
    o_ref[...] = (
        jnp.dot(s_ref[...], wm_ref[...], preferred_element_type=jnp.float32)
        + a_ref[...] * wa_ref[...]
        + b_ref[...]
    )


def _tc_linear(sums, amounts, wm, wa, b2):
    blk = 1024
    return pl.pallas_call(
        _lin_body,
        grid=(B // blk,),
        in_specs=[
            pl.BlockSpec((blk, D), lambda i: (i, 0)),
            pl.BlockSpec((blk, 1), lambda i: (i, 0)),
            pl.BlockSpec((D, NCAT), lambda i: (0, 0)),
            pl.BlockSpec((1, NCAT), lambda i: (0, 0)),
            pl.BlockSpec((1, NCAT), lambda i: (0, 0)),
        ],
        out_specs=pl.BlockSpec((blk, NCAT), lambda i: (i, 0)),
        out_shape=jax.ShapeDtypeStruct((B, NCAT), jnp.float32),
    )(sums, amounts, wm, wa, b2)


def kernel(descriptions, amounts, table, W, b):
    desc_flat = descriptions.reshape(-1)
    table_c = _tc_pack(table.T)
    sums = _sc_pool(desc_flat, table_c)
    wm = W[:D] * (1.0 / L)
    wa = W[D : D + 1]
    b2 = b.reshape(1, NCAT)
    return _tc_linear(sums, amounts, wm, wa, b2)
